# Initial kernel scaffold; baseline (speedup 1.0000x reference)
import jax
import jax.numpy as jnp
from jax.experimental import pallas as pl


def kernel(features, W1, W2, bias1, bias2, ln1_g, ln1_b, ln2_g, ln2_b, rows, cols, vals):
    raise NotImplementedError("write your pallas kernel here")


# speedup vs baseline: 11.6246x; 11.6246x over previous
import functools

import jax
import jax.numpy as jnp
from jax import lax
from jax.experimental import pallas as pl
from jax.experimental.pallas import tpu as pltpu
from jax.experimental.pallas import tpu_sc as plsc

N = 10000
RP = 17
EMB = 128
HID = 16
NCLS = 8

NC = 2
NS = 16
NW = NC * NS
BATCH = 128
RSTRIPE = 640
RTAIL = N - RSTRIPE * (NS - 1)


def _xw_body(x_ref, w_ref, o_ref):
    o_ref[0] = lax.dot_general(
        x_ref[...], w_ref[0], (((1,), (0,)), ((), ())),
        preferred_element_type=jnp.float32)


def _per_rel_transform(x, w):
    D, H = x.shape[1], w.shape[2]
    return pl.pallas_call(
        _xw_body,
        grid=(RP,),
        in_specs=[
            pl.BlockSpec((N, D), lambda r: (0, 0)),
            pl.BlockSpec((1, D, H), lambda r: (r, 0, 0)),
        ],
        out_specs=pl.BlockSpec((1, N, H), lambda r: (r, 0, 0)),
        out_shape=jax.ShapeDtypeStruct((RP, N, H), jnp.float32),
    )(x, w)


def _mid_body(p_ref, b_ref, g_ref, bb_ref, w_ref, o_ref):
    h = p_ref[0] + p_ref[1] + b_ref[...]
    mu = jnp.mean(h, axis=-1, keepdims=True)
    var = jnp.mean((h - mu) ** 2, axis=-1, keepdims=True)
    h = (h - mu) * lax.rsqrt(var + 1e-5) * g_ref[...] + bb_ref[...]
    h = jnp.maximum(h, 0.0)
    o_ref[0] = lax.dot_general(
        h, w_ref[0], (((1,), (0,)), ((), ())),
        preferred_element_type=jnp.float32)


def _mid_transform(partials, bias1, ln1_g, ln1_b, w2p):
    return pl.pallas_call(
        _mid_body,
        grid=(RP,),
        in_specs=[
            pl.BlockSpec((2, N, HID), lambda r: (0, 0, 0)),
            pl.BlockSpec((1, HID), lambda r: (0, 0)),
            pl.BlockSpec((1, HID), lambda r: (0, 0)),
            pl.BlockSpec((1, HID), lambda r: (0, 0)),
            pl.BlockSpec((1, HID, HID), lambda r: (r, 0, 0)),
        ],
        out_specs=pl.BlockSpec((1, N, HID), lambda r: (r, 0, 0)),
        out_shape=jax.ShapeDtypeStruct((RP, N, HID), jnp.float32),
    )(partials, bias1.reshape(1, HID), ln1_g.reshape(1, HID),
      ln1_b.reshape(1, HID), w2p)


def _fin_body(p_ref, b_ref, g_ref, bb_ref, o_ref):
    h = p_ref[0, :, :NCLS] + p_ref[1, :, :NCLS] + b_ref[...]
    mu = jnp.mean(h, axis=-1, keepdims=True)
    var = jnp.mean((h - mu) ** 2, axis=-1, keepdims=True)
    o_ref[...] = (h - mu) * lax.rsqrt(var + 1e-5) * g_ref[...] + bb_ref[...]


def _final_norm(partials, bias2, ln2_g, ln2_b):
    return pl.pallas_call(
        _fin_body,
        in_specs=[
            pl.BlockSpec((2, N, HID), lambda: (0, 0, 0)),
            pl.BlockSpec((1, NCLS), lambda: (0, 0)),
            pl.BlockSpec((1, NCLS), lambda: (0, 0)),
            pl.BlockSpec((1, NCLS), lambda: (0, 0)),
        ],
        out_specs=pl.BlockSpec((N, NCLS), lambda: (0, 0)),
        out_shape=jax.ShapeDtypeStruct((N, NCLS), jnp.float32),
    )(partials, bias2.reshape(1, NCLS), ln2_g.reshape(1, NCLS),
      ln2_b.reshape(1, NCLS))


def _make_edge_pass(e_pad):
    ce = e_pad // NW
    nb = ce // BATCH
    mesh = plsc.VectorSubcoreMesh(
        core_axis_name="c", subcore_axis_name="s",
        num_cores=NC, num_subcores=NS)

    @functools.partial(
        pl.kernel,
        out_type=jax.ShapeDtypeStruct((NC, N, HID), jnp.float32),
        mesh=mesh,
        compiler_params=pltpu.CompilerParams(use_tc_tiling_on_sc=False),
        scratch_types=[
            pltpu.VMEM((ce,), jnp.int32),
            pltpu.VMEM((ce,), jnp.int32),
            pltpu.VMEM((ce,), jnp.float32),
            pltpu.VMEM((nb, BATCH), jnp.int32),
            pltpu.VMEM((nb, BATCH), jnp.int32),
            pltpu.VMEM((BATCH, HID), jnp.float32),
            pltpu.VMEM_SHARED((N, HID), jnp.float32),
            pltpu.SemaphoreType.DMA,
        ],
    )
    def edge_pass(rows_hbm, cols_hbm, vals_hbm, table_hbm, zeros_hbm, out_hbm,
                  r_v, c_v, v_v, g_v, s_v, rb_v, acc_sh, sem):
        cid = lax.axis_index("c")
        sid = lax.axis_index("s")
        wid = sid * NC + cid
        base = wid * ce
        pltpu.sync_copy(rows_hbm.at[pl.ds(base, ce)], r_v)
        pltpu.sync_copy(cols_hbm.at[pl.ds(base, ce)], c_v)
        pltpu.sync_copy(vals_hbm.at[pl.ds(base, ce)], v_v)
        @pl.when(sid < NS - 1)
        def _():
            pltpu.sync_copy(zeros_hbm.at[pl.ds(sid * RSTRIPE, RSTRIPE)],
                            acc_sh.at[pl.ds(sid * RSTRIPE, RSTRIPE)])

        @pl.when(sid == NS - 1)
        def _():
            pltpu.sync_copy(zeros_hbm.at[pl.ds((NS - 1) * RSTRIPE, RTAIL)],
                            acc_sh.at[pl.ds((NS - 1) * RSTRIPE, RTAIL)])

        def ixbody(i, _):
            rr = r_v[pl.ds(i * 16, 16)]
            cc = c_v[pl.ds(i * 16, 16)]
            ss = lax.rem(rr, jnp.int32(N))
            gg = rr - ss + cc
            b = i // (BATCH // 16)
            o = (i % (BATCH // 16)) * 16
            g_v[b, pl.ds(o, 16)] = gg
            s_v[b, pl.ds(o, 16)] = ss
            return 0

        lax.fori_loop(0, ce // 16, ixbody, 0)
        plsc.subcore_barrier()

        def batch_body(b, _):
            pltpu.async_copy(table_hbm.at[g_v.at[b]], rb_v, sem).wait()

            def scale_body(i, _):
                v16 = v_v[pl.ds(b * BATCH + i * 16, 16)]
                for j in range(16):
                    k = i * 16 + j
                    rb_v[k, :] = rb_v[k, :] * v16[j]
                return 0

            lax.fori_loop(0, BATCH // 16, scale_body, 0)
            pltpu.sync_copy(rb_v, acc_sh.at[s_v.at[b]], add=True)
            return 0

        lax.fori_loop(0, nb, batch_body, 0)
        plsc.subcore_barrier()

        @pl.when(sid < NS - 1)
        def _():
            pltpu.sync_copy(acc_sh.at[pl.ds(sid * RSTRIPE, RSTRIPE)],
                            out_hbm.at[cid, pl.ds(sid * RSTRIPE, RSTRIPE)])

        @pl.when(sid == NS - 1)
        def _():
            pltpu.sync_copy(acc_sh.at[pl.ds((NS - 1) * RSTRIPE, RTAIL)],
                            out_hbm.at[cid, pl.ds((NS - 1) * RSTRIPE, RTAIL)])

    return edge_pass


def kernel(features, W1, W2, bias1, bias2, ln1_g, ln1_b, ln2_g, ln2_b,
           rows, cols, vals):
    e = rows.shape[0]
    e_pad = ((e + NW * BATCH - 1) // (NW * BATCH)) * (NW * BATCH)
    pad = e_pad - e
    rows_p = jnp.pad(rows.astype(jnp.int32), (0, pad))
    cols_p = jnp.pad(cols.astype(jnp.int32), (0, pad))
    vals_p = jnp.pad(vals, (0, pad))
    zeros_tab = jnp.zeros((N, HID), jnp.float32)
    w2p = jnp.pad(W2, ((0, 0), (0, 0), (0, HID - NCLS)))

    edge_pass = _make_edge_pass(e_pad)

    t1 = _per_rel_transform(features.astype(jnp.float32), W1)
    p1 = edge_pass(rows_p, cols_p, vals_p, t1.reshape(RP * N, HID), zeros_tab)
    t2 = _mid_transform(p1, bias1, ln1_g, ln1_b, w2p)
    p2 = edge_pass(rows_p, cols_p, vals_p, t2.reshape(RP * N, HID), zeros_tab)
    return _final_norm(p2, bias2, ln2_g, ln2_b)
